# baseline (speedup 1.0000x reference)
import functools

import jax
import jax.numpy as jnp
import numpy as np
from jax import lax
from jax.experimental import pallas as pl
from jax.experimental.pallas import tpu as pltpu
from jax.experimental.pallas import tpu_sc as plsc

SIZE = 384
PATCH = 16
NUM = SIZE // PATCH
C = 32
D = 768
P = NUM * NUM
NAUG = 8

_info = plsc.get_sparse_core_info()
NC, NS = _info.num_cores, _info.num_subcores
NW = NC * NS

TOTAL_ROWS = NAUG * C * P
K = 72
NCHUNK = P // K
SCATTER_AUGS = (1, 2, 3, 4, 6, 7)
NSA = len(SCATTER_AUGS)


def _build_tables():
    grid = np.arange(P, dtype=np.int32).reshape(NUM, NUM)
    srt_list = []
    for k in range(4):
        rg = np.rot90(grid, k=k, axes=(0, 1))
        srt_list.append(np.argsort(rg.flatten()))
        srt_list.append(np.argsort(np.flip(rg, axis=1).flatten()))
    srt = np.stack(srt_list).astype(np.int32)
    sel = srt[list(SCATTER_AUGS)]
    a_base = (np.asarray(SCATTER_AUGS, np.int32) * (C * P))[None, None, :,
                                                           None]
    c_base = (np.arange(C, dtype=np.int32) * P)[:, None, None, None]
    pos = sel.reshape(NSA, NCHUNK, K).transpose(1, 0, 2)[None]
    out_idx = a_base + c_base + pos
    return out_idx.reshape(C, NCHUNK * NSA, K).astype(np.int32), srt


_OUT_IDX_NP, _ARGSORT_NP = _build_tables()


@functools.partial(
    pl.kernel,
    mesh=plsc.VectorSubcoreMesh(core_axis_name="c", subcore_axis_name="s"),
    out_type=jax.ShapeDtypeStruct((TOTAL_ROWS, D), jnp.float32),
    scratch_types=[
        pltpu.VMEM((NCHUNK * NSA, K), jnp.int32),
        pltpu.VMEM((K, D), jnp.float32),
        pltpu.VMEM((K, D), jnp.float32),
        pltpu.SemaphoreType.DMA,
        pltpu.SemaphoreType.DMA,
        pltpu.SemaphoreType.DMA,
        pltpu.SemaphoreType.DMA,
        pltpu.SemaphoreType.DMA,
    ],
)
def _scatter_augs(pf_hbm, idx_hbm, out_hbm, idx_v, buf0, buf1, rs0, rs1,
                  ss0, ss1, cs):
    wid = lax.axis_index("s") * NC + lax.axis_index("c")
    pltpu.sync_copy(idx_hbm.at[wid], idx_v)
    src0 = pl.multiple_of(wid * P, P)

    def hbm_copies(run):
        run(pf_hbm.at[pl.ds(src0, P)],
            out_hbm.at[pl.ds(0 * C * P + src0, P)], cs)
        for r in range(NUM):
            run(pf_hbm.at[pl.ds(src0 + (NUM - 1 - r) * NUM, NUM)],
                out_hbm.at[pl.ds(5 * C * P + src0 + r * NUM, NUM)], cs)

    hbm_copies(lambda s, d, sem: pltpu.async_copy(s, d, sem))

    def read(t, buf, sem):
        pltpu.async_copy(pf_hbm.at[pl.ds(src0 + t * K, K)], buf, sem)

    def scatter_all(t, buf, sem):
        for a in range(NSA):
            pltpu.async_copy(buf, out_hbm.at[idx_v.at[t * NSA + a]], sem)

    def drain_all(t, buf, sem):
        for a in range(NSA):
            pltpu.make_async_copy(buf, out_hbm.at[idx_v.at[t * NSA + a]],
                                  sem).wait()

    read(0, buf0, rs0)
    read(1, buf1, rs1)

    def body(g, carry):
        t0 = 2 * g
        t1 = t0 + 1
        pltpu.make_async_copy(pf_hbm.at[pl.ds(src0 + t0 * K, K)], buf0,
                              rs0).wait()
        scatter_all(t0, buf0, ss0)
        pltpu.make_async_copy(pf_hbm.at[pl.ds(src0 + t1 * K, K)], buf1,
                              rs1).wait()
        scatter_all(t1, buf1, ss1)
        drain_all(t0, buf0, ss0)

        @pl.when(t0 + 2 < NCHUNK)
        def _():
            read(t0 + 2, buf0, rs0)

        drain_all(t1, buf1, ss1)

        @pl.when(t1 + 2 < NCHUNK)
        def _():
            read(t1 + 2, buf1, rs1)

        return carry

    lax.fori_loop(0, NCHUNK // 2, body, 0)
    hbm_copies(lambda s, d, sem: pltpu.make_async_copy(s, d, sem).wait())


def kernel(patch):
    pf = patch.reshape(C * P, D)
    idx = jnp.asarray(_OUT_IDX_NP)
    out_flat = _scatter_augs(pf, idx)
    aug = out_flat.reshape(NAUG, C, P, D)
    argsort = jnp.asarray(_ARGSORT_NP)
    perm = jnp.arange(NAUG, dtype=jnp.int32)
    return aug, argsort, perm


# speedup vs baseline: 18.1713x; 18.1713x over previous
import functools

import jax
import jax.numpy as jnp
import numpy as np
from jax import lax
from jax.experimental import pallas as pl
from jax.experimental.pallas import tpu as pltpu
from jax.experimental.pallas import tpu_sc as plsc

SIZE = 384
PATCH = 16
NUM = SIZE // PATCH
C = 32
D = 768
P = NUM * NUM
NAUG = 8

_info = plsc.get_sparse_core_info()
NC, NS = _info.num_cores, _info.num_subcores
NW = NC * NS

TOTAL_ROWS = NAUG * C * P
K = 72
NCHUNK = P // K


def _build_tables():
    grid = np.arange(P, dtype=np.int32).reshape(NUM, NUM)
    srt_list = []
    for k in range(4):
        rg = np.rot90(grid, k=k, axes=(0, 1))
        srt_list.append(np.argsort(rg.flatten()))
        srt_list.append(np.argsort(np.flip(rg, axis=1).flatten()))
    srt = np.stack(srt_list).astype(np.int32)
    a_base = (np.arange(NAUG, dtype=np.int32) * (C * P))[None, None, :, None]
    c_base = (np.arange(C, dtype=np.int32) * P)[:, None, None, None]
    pos = srt.reshape(NAUG, NCHUNK, K).transpose(1, 0, 2)[None]
    out_idx = a_base + c_base + pos
    return out_idx.reshape(C, NCHUNK * NAUG, K).astype(np.int32), srt


_OUT_IDX_NP, _ARGSORT_NP = _build_tables()


@functools.partial(
    pl.kernel,
    mesh=plsc.VectorSubcoreMesh(core_axis_name="c", subcore_axis_name="s"),
    out_type=jax.ShapeDtypeStruct((TOTAL_ROWS, D), jnp.float32),
    scratch_types=[
        pltpu.VMEM((NCHUNK * NAUG, K), jnp.int32),
        pltpu.VMEM((K, D), jnp.float32),
        pltpu.VMEM((K, D), jnp.float32),
        pltpu.SemaphoreType.DMA,
        pltpu.SemaphoreType.DMA,
        pltpu.SemaphoreType.DMA,
        pltpu.SemaphoreType.DMA,
    ],
)
def _scatter_augs(pf_hbm, idx_hbm, out_hbm, idx_v, buf0, buf1, rs0, rs1,
                  ss0, ss1):
    wid = lax.axis_index("s") * NC + lax.axis_index("c")
    pltpu.sync_copy(idx_hbm.at[wid], idx_v)
    src0 = pl.multiple_of(wid * P, P)

    def read(t, buf, sem):
        pltpu.async_copy(pf_hbm.at[pl.ds(src0 + t * K, K)], buf, sem)

    def scatter_all(t, buf, sem):
        for a in range(NAUG):
            pltpu.async_copy(buf, out_hbm.at[idx_v.at[t * NAUG + a]], sem)

    def drain_all(t, buf, sem):
        for a in range(NAUG):
            pltpu.make_async_copy(buf, out_hbm.at[idx_v.at[t * NAUG + a]],
                                  sem).wait()

    read(0, buf0, rs0)
    read(1, buf1, rs1)

    def body(g, carry):
        t0 = 2 * g
        t1 = t0 + 1
        pltpu.make_async_copy(pf_hbm.at[pl.ds(src0 + t0 * K, K)], buf0,
                              rs0).wait()
        scatter_all(t0, buf0, ss0)
        pltpu.make_async_copy(pf_hbm.at[pl.ds(src0 + t1 * K, K)], buf1,
                              rs1).wait()
        scatter_all(t1, buf1, ss1)
        drain_all(t0, buf0, ss0)

        @pl.when(t0 + 2 < NCHUNK)
        def _():
            read(t0 + 2, buf0, rs0)

        drain_all(t1, buf1, ss1)

        @pl.when(t1 + 2 < NCHUNK)
        def _():
            read(t1 + 2, buf1, rs1)

        return carry

    lax.fori_loop(0, NCHUNK // 2, body, 0)


def kernel(patch):
    pf = patch.reshape(C * P, D)
    idx = jnp.asarray(_OUT_IDX_NP)
    out_flat = _scatter_augs(pf, idx)
    aug = out_flat.reshape(NAUG, C, P, D)
    argsort = jnp.asarray(_ARGSORT_NP)
    perm = jnp.arange(NAUG, dtype=jnp.int32)
    return aug, argsort, perm
